# baseline (speedup 1.0000x reference)
import jax
import jax.numpy as jnp
from jax import lax
from jax.experimental import pallas as pl
from jax.experimental.pallas import tpu as pltpu
from jax.experimental.pallas import tpu_sc as plsc

_NUM_CLASSES = 1000
_N_CTX = 16
_DIM = 512
_SEQ = 77
_SUF = _SEQ - 1 - _N_CTX
_B = 4096

_NC = 2
_NS = 16
_NW = _NC * _NS
_K = 2
_BC = _B // _K
_BPW = _BC // _NW


def _sc_body(lab_hbm, aug_hbm, out_hbm, idx_v, buf, sem0, sem1):
    wid = lax.axis_index("s") * _NC + lax.axis_index("c")
    base = wid * _BPW

    pltpu.sync_copy(lab_hbm.at[wid], idx_v)
    sems = (sem0, sem1)

    def copy(c, b):
        return pltpu.make_async_copy(
            aug_hbm.at[idx_v.at[c]], buf.at[b], sems[b])

    copy(0, 0).start()
    copy(1, 1).start()

    def body(i, carry):
        for b in range(2):
            cur = 2 * i + b
            copy(cur, b).wait()
            pltpu.sync_copy(buf.at[b],
                            out_hbm.at[pl.ds(base + cur, 1), :, :, :])
            copy(jnp.minimum(cur + 2, _BPW - 1), b).start()
        return carry

    lax.fori_loop(0, _BPW // 2, body, 0)
    copy(_BPW - 1, 0).wait()
    copy(_BPW - 1, 1).wait()


@jax.jit
def kernel(label, cls_ctx, token_prefix, token_suffix):
    lab = label.astype(jnp.int32).reshape(_K, _NW, _BPW, 1)
    pre = jnp.broadcast_to(token_prefix, (_NUM_CLASSES, 1, _DIM))
    aug = jnp.concatenate([pre, cls_ctx, token_suffix], axis=1)
    aug4 = aug.reshape(_NUM_CLASSES, _SEQ, 1, _DIM)

    mesh = plsc.VectorSubcoreMesh(core_axis_name="c", subcore_axis_name="s")
    sc_gather = pl.kernel(
        _sc_body,
        out_type=jax.ShapeDtypeStruct((_BC, _SEQ, 1, _DIM), jnp.float32),
        mesh=mesh,
        scratch_types=[
            pltpu.VMEM((_BPW, 1), jnp.int32),
            pltpu.VMEM((2, 1, _SEQ, 1, _DIM), jnp.float32),
            pltpu.SemaphoreType.DMA,
            pltpu.SemaphoreType.DMA,
        ],
    )
    chunks = [
        sc_gather(lab[k], aug4).reshape(_BC, _SEQ, _DIM) for k in range(_K)
    ]
    return jnp.concatenate(chunks, axis=0)


# speedup vs baseline: 1.3766x; 1.3766x over previous
import jax
import jax.numpy as jnp
from jax import lax
from jax.experimental import pallas as pl
from jax.experimental.pallas import tpu as pltpu
from jax.experimental.pallas import tpu_sc as plsc

_NUM_CLASSES = 1000
_N_CTX = 16
_DIM = 512
_SEQ = 77
_SUF = _SEQ - 1 - _N_CTX
_B = 4096

_NC = 2
_NS = 16
_NW = _NC * _NS
_BPW = _B // _NW


def _sc_body(lab_hbm, aug_hbm, out_hbm, idx_v, buf, sem0, sem1):
    wid = lax.axis_index("s") * _NC + lax.axis_index("c")
    base = wid * _BPW

    pltpu.sync_copy(lab_hbm.at[wid], idx_v)
    sems = (sem0, sem1)

    def copy(c, b):
        return pltpu.make_async_copy(
            aug_hbm.at[idx_v.at[c]], buf.at[b], sems[b])

    copy(0, 0).start()
    copy(1, 1).start()

    def body(i, carry):
        for b in range(2):
            cur = 2 * i + b
            copy(cur, b).wait()
            pltpu.sync_copy(buf.at[b],
                            out_hbm.at[pl.ds(base + cur, 1), :, :, :])
            copy(jnp.minimum(cur + 2, _BPW - 1), b).start()
        return carry

    lax.fori_loop(0, _BPW // 2, body, 0)
    copy(_BPW - 1, 0).wait()
    copy(_BPW - 1, 1).wait()


@jax.jit
def kernel(label, cls_ctx, token_prefix, token_suffix):
    lab = label.astype(jnp.int32).reshape(_NW, _BPW, 1)
    pre4 = jnp.broadcast_to(token_prefix.reshape(1, 1, 1, _DIM),
                            (_NUM_CLASSES, 1, 1, _DIM))
    ctx4 = cls_ctx.reshape(_NUM_CLASSES, _N_CTX, 1, _DIM)
    suf4 = token_suffix.reshape(_NUM_CLASSES, _SUF, 1, _DIM)
    aug4 = jnp.concatenate([pre4, ctx4, suf4], axis=1)

    mesh = plsc.VectorSubcoreMesh(core_axis_name="c", subcore_axis_name="s")
    out = pl.kernel(
        _sc_body,
        out_type=jax.ShapeDtypeStruct((_B, _SEQ, 1, _DIM), jnp.float32),
        mesh=mesh,
        scratch_types=[
            pltpu.VMEM((_BPW, 1), jnp.int32),
            pltpu.VMEM((2, 1, _SEQ, 1, _DIM), jnp.float32),
            pltpu.SemaphoreType.DMA,
            pltpu.SemaphoreType.DMA,
        ],
    )(lab, aug4)
    return out.reshape(_B, _SEQ, _DIM)
